# Initial kernel scaffold; baseline (speedup 1.0000x reference)
import jax
import jax.numpy as jnp
from jax.experimental import pallas as pl


def kernel(inputs, token_table, pos_table):
    raise NotImplementedError("write your pallas kernel here")


# speedup vs baseline: 1.2763x; 1.2763x over previous
import functools
import jax
import jax.numpy as jnp
from jax import lax
from jax.experimental import pallas as pl
from jax.experimental.pallas import tpu as pltpu
from jax.experimental.pallas import tpu_sc as plsc

SEQ = 2048
DIM = 128
NB = 4

_info = plsc.get_sparse_core_info()
_NC = _info.num_cores
_NS = _info.num_subcores
_L = _info.num_lanes
NW = _NC * _NS
BTOT = NB * SEQ
BPW = BTOT // NW
ICH = 128
NCH = BPW // ICH


def _sc_body(idx_hbm, tok_hbm, pos_hbm, out_hbm, idx_v, rows_v, pos_v, gsem, psem):
    wid = lax.axis_index("s") * _NC + lax.axis_index("c")
    base = wid * BPW
    s0 = base % SEQ

    pltpu.sync_copy(idx_hbm.at[wid], idx_v)
    gathers = [
        pltpu.async_copy(
            tok_hbm.at[idx_v.at[j]], rows_v.at[pl.ds(j * ICH, ICH)], gsem
        )
        for j in range(NCH)
    ]
    pcopy = pltpu.async_copy(pos_hbm.at[pl.ds(s0, BPW)], pos_v, psem)
    for g in gathers:
        g.wait()
    pcopy.wait()

    def add_row(i, carry):
        for c in range(DIM // _L):
            sl = pl.ds(c * _L, _L)
            rows_v[i, sl] = rows_v[i, sl] + pos_v[i, sl]
        return carry

    lax.fori_loop(0, BPW, add_row, 0)

    pltpu.sync_copy(rows_v, out_hbm.at[pl.ds(base, BPW)])


@functools.partial(jax.jit, static_argnums=())
def _sc_embed(idx, token_table, pos_table):
    kern = pl.kernel(
        _sc_body,
        out_type=jax.ShapeDtypeStruct((BTOT, DIM), jnp.float32),
        mesh=plsc.VectorSubcoreMesh(core_axis_name="c", subcore_axis_name="s"),
        scratch_types=[
            pltpu.VMEM((NCH, ICH), jnp.int32),
            pltpu.VMEM((BPW, DIM), jnp.float32),
            pltpu.VMEM((BPW, DIM), jnp.float32),
            pltpu.SemaphoreType.DMA,
            pltpu.SemaphoreType.DMA,
        ],
    )
    return kern(idx, token_table, pos_table)


def kernel(inputs, token_table, pos_table):
    idx = inputs.astype(jnp.int32).reshape(NW, NCH, ICH)
    out = _sc_embed(idx, token_table, pos_table)
    return out.reshape(NB, SEQ, DIM)


# speedup vs baseline: 1.3316x; 1.0434x over previous
import functools
import jax
import jax.numpy as jnp
from jax import lax
from jax.experimental import pallas as pl
from jax.experimental.pallas import tpu as pltpu
from jax.experimental.pallas import tpu_sc as plsc

SEQ = 2048
DIM = 128
NB = 4

_info = plsc.get_sparse_core_info()
_NC = _info.num_cores
_NS = _info.num_subcores
_L = _info.num_lanes
NW = _NC * _NS
BTOT = NB * SEQ
BPW = BTOT // NW
SPW = SEQ // (NW // NB)
RB = 64
NBLK = BPW // RB


def _sc_body(idx_hbm, tok_hbm, pos_hbm, out_hbm, idx_v, rows_v, pos_v,
             gsems, psems, osem):
    wid = lax.axis_index("s") * _NC + lax.axis_index("c")
    base = wid * BPW
    b = wid // (NW // NB)
    c0 = (wid % (NW // NB)) * BPW
    s0 = base % SEQ

    pltpu.sync_copy(idx_hbm.at[b, pl.ds(c0, BPW)], idx_v)
    gathers = []
    pcopies = []
    for j in range(NBLK):
        gathers.append(
            pltpu.async_copy(
                tok_hbm.at[idx_v.at[pl.ds(j * RB, RB)]],
                rows_v.at[pl.ds(j * RB, RB)],
                gsems.at[j],
            )
        )
        pcopies.append(
            pltpu.async_copy(
                pos_hbm.at[pl.ds(s0 + j * RB, RB)],
                pos_v.at[pl.ds(j * RB, RB)],
                psems.at[j],
            )
        )

    outs = []
    for j in range(NBLK):
        gathers[j].wait()
        pcopies[j].wait()

        def add_row(i, carry):
            for c in range(DIM // _L):
                sl = pl.ds(c * _L, _L)
                rows_v[i, sl] = rows_v[i, sl] + pos_v[i, sl]
            return carry

        lax.fori_loop(j * RB, (j + 1) * RB, add_row, 0)
        outs.append(
            pltpu.async_copy(
                rows_v.at[pl.ds(j * RB, RB)],
                out_hbm.at[pl.ds(base + j * RB, RB)],
                osem,
            )
        )
    for o in outs:
        o.wait()


@jax.jit
def _sc_embed(idx, token_table, pos_table):
    kern = pl.kernel(
        _sc_body,
        out_type=jax.ShapeDtypeStruct((BTOT, DIM), jnp.float32),
        mesh=plsc.VectorSubcoreMesh(core_axis_name="c", subcore_axis_name="s"),
        scratch_types=[
            pltpu.VMEM((BPW,), jnp.int32),
            pltpu.VMEM((BPW, DIM), jnp.float32),
            pltpu.VMEM((BPW, DIM), jnp.float32),
            pltpu.SemaphoreType.DMA((NBLK,)),
            pltpu.SemaphoreType.DMA((NBLK,)),
            pltpu.SemaphoreType.DMA,
        ],
    )
    return kern(idx, token_table, pos_table)


def kernel(inputs, token_table, pos_table):
    out = _sc_embed(inputs.astype(jnp.int32), token_table, pos_table)
    return out.reshape(NB, SEQ, DIM)


# speedup vs baseline: 1.3841x; 1.0394x over previous
import jax
import jax.numpy as jnp
from jax import lax
from jax.experimental import pallas as pl
from jax.experimental.pallas import tpu as pltpu
from jax.experimental.pallas import tpu_sc as plsc

SEQ = 2048
DIM = 128
NB = 4

_info = plsc.get_sparse_core_info()
_NC = _info.num_cores
_NS = _info.num_subcores
_L = _info.num_lanes
NW = _NC * _NS
SPW = SEQ // NW
BPW = NB * SPW


def _sc_body(idx_hbm, tok_hbm, pos_hbm, out_hbm, idx_v, rows_v, pos_v,
             isems, gsems, psem, osem):
    wid = lax.axis_index("s") * _NC + lax.axis_index("c")
    s0 = wid * SPW

    icopies = [
        pltpu.async_copy(idx_hbm.at[b, pl.ds(s0, SPW)], idx_v.at[b],
                         isems.at[b])
        for b in range(NB)
    ]
    pcopy = pltpu.async_copy(pos_hbm.at[pl.ds(s0, SPW)], pos_v, psem)
    gathers = []
    for b in range(NB):
        icopies[b].wait()
        gathers.append(
            pltpu.async_copy(tok_hbm.at[idx_v.at[b]],
                             rows_v.at[pl.ds(b * SPW, SPW)], gsems.at[b])
        )

    outs = []
    for b in range(NB):
        gathers[b].wait()
        if b == 0:
            pcopy.wait()
        r0 = b * SPW

        def add_row(i, carry):
            for c in range(DIM // _L):
                sl = pl.ds(c * _L, _L)
                rows_v[r0 + i, sl] = rows_v[r0 + i, sl] + pos_v[i, sl]
            return carry

        lax.fori_loop(0, SPW, add_row, 0)
        outs.append(
            pltpu.async_copy(rows_v.at[pl.ds(r0, SPW)],
                             out_hbm.at[pl.ds(b * SEQ + s0, SPW)], osem)
        )
    for o in outs:
        o.wait()


@jax.jit
def _sc_embed(idx, token_table, pos_table):
    kern = pl.kernel(
        _sc_body,
        out_type=jax.ShapeDtypeStruct((NB * SEQ, DIM), jnp.float32),
        mesh=plsc.VectorSubcoreMesh(core_axis_name="c", subcore_axis_name="s"),
        scratch_types=[
            pltpu.VMEM((NB, SPW), jnp.int32),
            pltpu.VMEM((BPW, DIM), jnp.float32),
            pltpu.VMEM((SPW, DIM), jnp.float32),
            pltpu.SemaphoreType.DMA((NB,)),
            pltpu.SemaphoreType.DMA((NB,)),
            pltpu.SemaphoreType.DMA,
            pltpu.SemaphoreType.DMA,
        ],
    )
    return kern(idx, token_table, pos_table)


def kernel(inputs, token_table, pos_table):
    out = _sc_embed(inputs.astype(jnp.int32), token_table, pos_table)
    return out.reshape(NB, SEQ, DIM)
